# Initial kernel scaffold; baseline (speedup 1.0000x reference)
import jax
import jax.numpy as jnp
from jax.experimental import pallas as pl


def kernel(x, edge_index, W1, b1, W2, b2, Wr, br):
    raise NotImplementedError("write your pallas kernel here")


# speedup vs baseline: 7.3914x; 7.3914x over previous
import functools

import jax
import jax.numpy as jnp
from jax import lax
from jax.experimental import pallas as pl
from jax.experimental.pallas import tpu as pltpu
from jax.experimental.pallas import tpu_sc as plsc

N_NODES = 10000
N_PAD = 10240
E_EDGES = 160000
CHUNK = 128
E_PAD = 161792
D_IN = 256
D_HID = 256
D_OUT = 128

_MESH = plsc.VectorSubcoreMesh(
    core_axis_name="c", subcore_axis_name="s", num_cores=2, num_subcores=16
)


@functools.partial(
    pl.kernel,
    out_type=jax.ShapeDtypeStruct((N_PAD, 8), jnp.float32),
    mesh=_MESH,
    scratch_types=[
        pltpu.VMEM((CHUNK, 8), jnp.float32),
        pltpu.VMEM((CHUNK,), jnp.int32),
        pltpu.VMEM_SHARED((N_PAD, 8), jnp.float32),
    ],
)
def _deg_kernel(ones_hbm, dst_hbm, out_hbm, ones_v, idx_v, acc_sh):
    c = lax.axis_index("c")
    s = lax.axis_index("s")
    pltpu.sync_copy(ones_hbm, ones_v)
    rpt = N_PAD // 16
    for r in range(rpt // CHUNK):
        pltpu.sync_copy(ones_v, acc_sh.at[pl.ds(s * rpt + r * CHUNK, CHUNK)])
    plsc.subcore_barrier()

    ept = E_PAD // 16

    def body(k, carry):
        pltpu.sync_copy(dst_hbm.at[pl.ds(s * ept + k * CHUNK, CHUNK)], idx_v)
        pltpu.sync_copy(ones_v, acc_sh.at[idx_v], add=True)
        return carry

    lax.fori_loop(0, ept // CHUNK, body, 0)
    plsc.subcore_barrier()
    orow = N_PAD // 32
    ob = c * (N_PAD // 2) + s * orow
    pltpu.sync_copy(acc_sh.at[pl.ds(ob, orow)], out_hbm.at[pl.ds(ob, orow)])


@functools.partial(
    pl.kernel,
    out_type=jax.ShapeDtypeStruct((2 * N_PAD, 128), jnp.float32),
    mesh=_MESH,
    scratch_types=[
        pltpu.VMEM((CHUNK,), jnp.int32),
        pltpu.VMEM((CHUNK,), jnp.int32),
        pltpu.VMEM((CHUNK, 128), jnp.float32),
        pltpu.VMEM_SHARED((N_PAD, 128), jnp.float32),
        pltpu.SemaphoreType.DMA,
    ],
)
def _agg_kernel(table_hbm, src2_hbm, dst_hbm, out_hbm, sidx, didx, rows, acc_sh, sem):
    c = lax.axis_index("c")
    s = lax.axis_index("s")
    rpt = N_PAD // 16
    b = s * rpt
    pltpu.sync_copy(table_hbm.at[pl.ds(c * N_PAD + b, rpt)], acc_sh.at[pl.ds(b, rpt)])
    plsc.subcore_barrier()

    ept = E_PAD // 16

    def body(k, carry):
        e0 = s * ept + k * CHUNK
        pltpu.sync_copy(src2_hbm.at[pl.ds(c * E_PAD + e0, CHUNK)], sidx)
        pltpu.sync_copy(dst_hbm.at[pl.ds(e0, CHUNK)], didx)
        pltpu.async_copy(table_hbm.at[sidx], rows, sem).wait()
        pltpu.sync_copy(rows, acc_sh.at[didx], add=True)
        return carry

    lax.fori_loop(0, ept // CHUNK, body, 0)
    plsc.subcore_barrier()
    pltpu.sync_copy(acc_sh.at[pl.ds(b, rpt)], out_hbm.at[pl.ds(c * N_PAD + b, rpt)])


_RB = 1024
_NRB = N_PAD // _RB


def _lin1_body(x_ref, w_ref, deg_ref, tab_ref, dinv_ref):
    di8 = lax.rsqrt(deg_ref[...])
    dinv_ref[...] = di8
    h = jnp.dot(x_ref[...], w_ref[...], preferred_element_type=jnp.float32)
    tab_ref[...] = h * di8[:, :1]


def _lin1(x_pad, w1, deg8):
    return pl.pallas_call(
        _lin1_body,
        grid=(_NRB, 2),
        in_specs=[
            pl.BlockSpec((_RB, D_IN), lambda i, j: (i, 0)),
            pl.BlockSpec((D_IN, 128), lambda i, j: (0, j)),
            pl.BlockSpec((_RB, 8), lambda i, j: (i, 0)),
        ],
        out_specs=[
            pl.BlockSpec((_RB, 128), lambda i, j: (j * _NRB + i, 0)),
            pl.BlockSpec((_RB, 8), lambda i, j: (i, 0)),
        ],
        out_shape=[
            jax.ShapeDtypeStruct((2 * N_PAD, 128), jnp.float32),
            jax.ShapeDtypeStruct((N_PAD, 8), jnp.float32),
        ],
    )(x_pad, w1, deg8)


def _mid_body(agg_ref, dinv_ref, b_ref, w_ref, out_ref):
    di = dinv_ref[:, :1]
    full = jnp.concatenate([agg_ref[0], agg_ref[1]], axis=1)
    h = jnp.maximum(full * di + b_ref[0:1, :], 0.0)
    out_ref[...] = jnp.dot(h, w_ref[...], preferred_element_type=jnp.float32) * di


def _mid(agg, dinv8, b8, w2):
    return pl.pallas_call(
        _mid_body,
        grid=(_NRB, 2),
        in_specs=[
            pl.BlockSpec((2, _RB, 128), lambda i, j: (0, i, 0)),
            pl.BlockSpec((_RB, 8), lambda i, j: (i, 0)),
            pl.BlockSpec((8, D_HID), lambda i, j: (0, 0)),
            pl.BlockSpec((D_HID, 128), lambda i, j: (0, j)),
        ],
        out_specs=pl.BlockSpec((_RB, 128), lambda i, j: (j * _NRB + i, 0)),
        out_shape=jax.ShapeDtypeStruct((2 * N_PAD, 128), jnp.float32),
    )(agg, dinv8, b8, w2)


def _head_body(agg_ref, dinv_ref, b_ref, wr_ref, br_ref, out_ref):
    di = dinv_ref[:, :1]
    full = jnp.concatenate([agg_ref[0], agg_ref[1]], axis=1)
    h = jnp.maximum(full * di + b_ref[0:1, :], 0.0)
    lg = jnp.dot(h, wr_ref[...], preferred_element_type=jnp.float32) + br_ref[0:1, :]
    m = jnp.max(lg, axis=1, keepdims=True)
    e = jnp.exp(lg - m)
    out_ref[...] = e / jnp.sum(e, axis=1, keepdims=True)


def _head(agg, dinv8, b8, wr, br8):
    return pl.pallas_call(
        _head_body,
        grid=(_NRB,),
        in_specs=[
            pl.BlockSpec((2, _RB, 128), lambda i: (0, i, 0)),
            pl.BlockSpec((_RB, 8), lambda i: (i, 0)),
            pl.BlockSpec((8, D_HID), lambda i: (0, 0)),
            pl.BlockSpec((D_HID, D_OUT), lambda i: (0, 0)),
            pl.BlockSpec((8, D_OUT), lambda i: (0, 0)),
        ],
        out_specs=pl.BlockSpec((_RB, D_OUT), lambda i: (i, 0)),
        out_shape=jax.ShapeDtypeStruct((N_PAD, D_OUT), jnp.float32),
    )(agg, dinv8, b8, wr, br8)


@jax.jit
def _run(x, edge_index, w1, b1, w2, b2, wr, br):
    src = edge_index[0]
    dst = edge_index[1]
    pad = E_PAD - E_EDGES
    srcp = jnp.concatenate([src, jnp.zeros((pad,), jnp.int32)])
    dstp = jnp.concatenate([dst, jnp.full((pad,), N_NODES, jnp.int32)])
    src2 = jnp.concatenate([srcp, srcp + N_PAD])

    ones8 = jnp.ones((CHUNK, 8), jnp.float32)
    deg8 = _deg_kernel(ones8, dstp)
    dst2 = jnp.concatenate([dstp, dstp + N_PAD])

    x_pad = jnp.concatenate([x, jnp.zeros((N_PAD - N_NODES, D_IN), jnp.float32)])
    tab1, dinv8 = _lin1(x_pad, w1, deg8)

    agg1 = _agg_kernel(tab1, src2, dstp).reshape(2, N_PAD, 128)

    b1_8 = jnp.broadcast_to(b1, (8, D_HID))
    tab2 = _mid(agg1, dinv8, b1_8, w2)

    agg2 = _agg_kernel(tab2, src2, dstp).reshape(2, N_PAD, 128)

    b2_8 = jnp.broadcast_to(b2, (8, D_HID))
    br_8 = jnp.broadcast_to(br, (8, D_OUT))
    out = _head(agg2, dinv8, b2_8, wr, br_8)
    return out[:N_NODES]


def kernel(x, edge_index, W1, b1, W2, b2, Wr, br):
    return _run(x, edge_index, W1, b1, W2, b2, Wr, br)


# speedup vs baseline: 7.7303x; 1.0458x over previous
import functools

import jax
import jax.numpy as jnp
from jax import lax
from jax.experimental import pallas as pl
from jax.experimental.pallas import tpu as pltpu
from jax.experimental.pallas import tpu_sc as plsc

N_NODES = 10000
N_PAD = 10240
E_EDGES = 160000
CHUNK = 128
GRP = 5
CPT = 80
E_PAD = 16 * CPT * CHUNK
E_ROWS = E_PAD // CHUNK
D_IN = 256
D_HID = 256
D_OUT = 128

_MESH = plsc.VectorSubcoreMesh(
    core_axis_name="c", subcore_axis_name="s", num_cores=2, num_subcores=16
)


@functools.partial(
    pl.kernel,
    out_type=jax.ShapeDtypeStruct((N_PAD, 8), jnp.float32),
    mesh=_MESH,
    scratch_types=[
        pltpu.VMEM((CHUNK, 8), jnp.float32),
        pltpu.VMEM((CPT, CHUNK), jnp.int32),
        pltpu.VMEM_SHARED((N_PAD, 8), jnp.float32),
        pltpu.SemaphoreType.DMA,
    ],
)
def _deg_kernel(ones_hbm, dst2d_hbm, out_hbm, ones_v, didx, acc_sh, sem):
    c = lax.axis_index("c")
    s = lax.axis_index("s")
    pltpu.sync_copy(ones_hbm, ones_v)
    pltpu.sync_copy(dst2d_hbm.at[pl.ds(s * CPT, CPT)], didx)
    rpt = N_PAD // 16
    for r in range(rpt // CHUNK):
        pltpu.sync_copy(ones_v, acc_sh.at[pl.ds(s * rpt + r * CHUNK, CHUNK)])
    plsc.subcore_barrier()

    def body(g, carry):
        ds = [
            pltpu.async_copy(ones_v, acc_sh.at[didx.at[g * 10 + j]], sem, add=True)
            for j in range(10)
        ]
        for d in ds:
            d.wait()
        return carry

    lax.fori_loop(0, CPT // 10, body, 0)
    plsc.subcore_barrier()
    orow = N_PAD // 32
    ob = c * (N_PAD // 2) + s * orow
    pltpu.sync_copy(acc_sh.at[pl.ds(ob, orow)], out_hbm.at[pl.ds(ob, orow)])


@functools.partial(
    pl.kernel,
    out_type=jax.ShapeDtypeStruct((2 * N_PAD, 128), jnp.float32),
    mesh=_MESH,
    scratch_types=[
        pltpu.VMEM((CPT, CHUNK), jnp.int32),
        pltpu.VMEM((4, CHUNK), jnp.int32),
        pltpu.VMEM((2, CHUNK, 128), jnp.float32),
        pltpu.VMEM_SHARED((N_PAD, 128), jnp.float32),
        pltpu.SemaphoreType.DMA,
        pltpu.SemaphoreType.DMA,
        pltpu.SemaphoreType.DMA,
    ],
)
def _agg_kernel(
    table_hbm, src1_hbm, dst2d_hbm, out_hbm, didx, sidxr, rows, acc_sh,
    isem, gsem, ssem,
):
    c = lax.axis_index("c")
    s = lax.axis_index("s")
    rpt = N_PAD // 16
    b = s * rpt
    eb = c * E_PAD + s * (CPT * CHUNK)
    pltpu.sync_copy(table_hbm.at[pl.ds(c * N_PAD + b, rpt)], acc_sh.at[pl.ds(b, rpt)])
    pltpu.sync_copy(dst2d_hbm.at[pl.ds(s * CPT, CPT)], didx)
    pltpu.async_copy(src1_hbm.at[pl.ds(eb, CHUNK)], sidxr.at[0], isem)
    pltpu.async_copy(src1_hbm.at[pl.ds(eb + CHUNK, CHUNK)], sidxr.at[1], isem)
    plsc.subcore_barrier()
    pltpu.make_async_copy(src1_hbm.at[pl.ds(eb, CHUNK)], sidxr.at[0], isem).wait()
    pltpu.async_copy(table_hbm.at[sidxr.at[0]], rows.at[0], gsem)

    def slot(k, carry):
        pp = lax.rem(k, 2)
        np_ = 1 - pp

        @pl.when(k + 1 < CPT)
        def _():
            pltpu.make_async_copy(
                src1_hbm.at[pl.ds(eb, CHUNK)], sidxr.at[0], isem
            ).wait()

        pltpu.make_async_copy(
            table_hbm.at[pl.ds(0, CHUNK)], rows.at[pp], gsem
        ).wait()
        pltpu.async_copy(rows.at[pp], acc_sh.at[didx.at[k]], ssem, add=True)

        @pl.when(k >= 1)
        def _():
            pltpu.make_async_copy(
                rows.at[np_], acc_sh.at[didx.at[k]], ssem
            ).wait()

        @pl.when(k + 2 < CPT)
        def _():
            pltpu.async_copy(
                src1_hbm.at[pl.ds(eb + (k + 2) * CHUNK, CHUNK)],
                sidxr.at[lax.rem(k + 2, 4)],
                isem,
            )

        @pl.when(k + 1 < CPT)
        def _():
            pltpu.async_copy(
                table_hbm.at[sidxr.at[lax.rem(k + 1, 4)]], rows.at[np_], gsem
            )

        return carry

    lax.fori_loop(0, CPT, slot, 0)
    pltpu.make_async_copy(
        rows.at[lax.rem(CPT - 1, 2)], acc_sh.at[didx.at[0]], ssem
    ).wait()
    plsc.subcore_barrier()
    pltpu.sync_copy(acc_sh.at[pl.ds(b, rpt)], out_hbm.at[pl.ds(c * N_PAD + b, rpt)])


_RB = 1024
_NRB = N_PAD // _RB


def _lin1_body(x_ref, w_ref, deg_ref, tab_ref, dinv_ref):
    di8 = lax.rsqrt(deg_ref[...])
    dinv_ref[...] = di8
    h = jnp.dot(x_ref[...], w_ref[...], preferred_element_type=jnp.float32)
    tab_ref[...] = h * di8[:, :1]


def _lin1(x_pad, w1, deg8):
    return pl.pallas_call(
        _lin1_body,
        grid=(_NRB, 2),
        in_specs=[
            pl.BlockSpec((_RB, D_IN), lambda i, j: (i, 0)),
            pl.BlockSpec((D_IN, 128), lambda i, j: (0, j)),
            pl.BlockSpec((_RB, 8), lambda i, j: (i, 0)),
        ],
        out_specs=[
            pl.BlockSpec((_RB, 128), lambda i, j: (j * _NRB + i, 0)),
            pl.BlockSpec((_RB, 8), lambda i, j: (i, 0)),
        ],
        out_shape=[
            jax.ShapeDtypeStruct((2 * N_PAD, 128), jnp.float32),
            jax.ShapeDtypeStruct((N_PAD, 8), jnp.float32),
        ],
    )(x_pad, w1, deg8)


def _mid_body(agg_ref, dinv_ref, b_ref, w_ref, out_ref):
    di = dinv_ref[:, :1]
    full = jnp.concatenate([agg_ref[0], agg_ref[1]], axis=1)
    h = jnp.maximum(full * di + b_ref[0:1, :], 0.0)
    out_ref[...] = jnp.dot(h, w_ref[...], preferred_element_type=jnp.float32) * di


def _mid(agg, dinv8, b8, w2):
    return pl.pallas_call(
        _mid_body,
        grid=(_NRB, 2),
        in_specs=[
            pl.BlockSpec((2, _RB, 128), lambda i, j: (0, i, 0)),
            pl.BlockSpec((_RB, 8), lambda i, j: (i, 0)),
            pl.BlockSpec((8, D_HID), lambda i, j: (0, 0)),
            pl.BlockSpec((D_HID, 128), lambda i, j: (0, j)),
        ],
        out_specs=pl.BlockSpec((_RB, 128), lambda i, j: (j * _NRB + i, 0)),
        out_shape=jax.ShapeDtypeStruct((2 * N_PAD, 128), jnp.float32),
    )(agg, dinv8, b8, w2)


def _head_body(agg_ref, dinv_ref, b_ref, wr_ref, br_ref, out_ref):
    di = dinv_ref[:, :1]
    full = jnp.concatenate([agg_ref[0], agg_ref[1]], axis=1)
    h = jnp.maximum(full * di + b_ref[0:1, :], 0.0)
    lg = jnp.dot(h, wr_ref[...], preferred_element_type=jnp.float32) + br_ref[0:1, :]
    m = jnp.max(lg, axis=1, keepdims=True)
    e = jnp.exp(lg - m)
    out_ref[...] = e / jnp.sum(e, axis=1, keepdims=True)


def _head(agg, dinv8, b8, wr, br8):
    return pl.pallas_call(
        _head_body,
        grid=(_NRB,),
        in_specs=[
            pl.BlockSpec((2, _RB, 128), lambda i: (0, i, 0)),
            pl.BlockSpec((_RB, 8), lambda i: (i, 0)),
            pl.BlockSpec((8, D_HID), lambda i: (0, 0)),
            pl.BlockSpec((D_HID, D_OUT), lambda i: (0, 0)),
            pl.BlockSpec((8, D_OUT), lambda i: (0, 0)),
        ],
        out_specs=pl.BlockSpec((_RB, D_OUT), lambda i: (i, 0)),
        out_shape=jax.ShapeDtypeStruct((N_PAD, D_OUT), jnp.float32),
    )(agg, dinv8, b8, wr, br8)


@jax.jit
def _run(x, edge_index, w1, b1, w2, b2, wr, br):
    src = edge_index[0]
    dst = edge_index[1]
    pad = E_PAD - E_EDGES
    srcp = jnp.concatenate([src, jnp.zeros((pad,), jnp.int32)])
    dstp = jnp.concatenate([dst, jnp.full((pad,), N_NODES, jnp.int32)])
    src1 = jnp.concatenate([srcp, srcp + N_PAD])
    dst2d = dstp.reshape(E_ROWS, CHUNK)

    ones8 = jnp.ones((CHUNK, 8), jnp.float32)
    deg8 = _deg_kernel(ones8, dst2d)

    x_pad = jnp.concatenate([x, jnp.zeros((N_PAD - N_NODES, D_IN), jnp.float32)])
    tab1, dinv8 = _lin1(x_pad, w1, deg8)

    agg1 = _agg_kernel(tab1, src1, dst2d).reshape(2, N_PAD, 128)

    b1_8 = jnp.broadcast_to(b1, (8, D_HID))
    tab2 = _mid(agg1, dinv8, b1_8, w2)

    agg2 = _agg_kernel(tab2, src1, dst2d).reshape(2, N_PAD, 128)

    b2_8 = jnp.broadcast_to(b2, (8, D_HID))
    br_8 = jnp.broadcast_to(br, (8, D_OUT))
    out = _head(agg2, dinv8, b2_8, wr, br_8)
    return out[:N_NODES]


def kernel(x, edge_index, W1, b1, W2, b2, Wr, br):
    return _run(x, edge_index, W1, b1, W2, b2, Wr, br)


# speedup vs baseline: 14.8423x; 1.9200x over previous
import functools

import jax
import jax.numpy as jnp
from jax import lax
from jax.experimental import pallas as pl
from jax.experimental.pallas import tpu as pltpu
from jax.experimental.pallas import tpu_sc as plsc

N_NODES = 10000
N_PAD = 10240
E_EDGES = 160000
CHUNK = 120
CPT = 84
E_PAD = 16 * CPT * CHUNK
ACC_ROWS = 10112
ACC_RPT = ACC_ROWS // 16
DEG_CPT = 80
DEG_E_PAD = 16 * DEG_CPT * 128
DEG_E_ROWS = DEG_E_PAD // 128
D_IN = 256
D_HID = 256
D_OUT = 128

_MESH = plsc.VectorSubcoreMesh(
    core_axis_name="c", subcore_axis_name="s", num_cores=2, num_subcores=16
)


@functools.partial(
    pl.kernel,
    out_type=jax.ShapeDtypeStruct((N_PAD, 8), jnp.float32),
    mesh=_MESH,
    scratch_types=[
        pltpu.VMEM((128, 8), jnp.float32),
        pltpu.VMEM((DEG_CPT, 128), jnp.int32),
        pltpu.VMEM_SHARED((N_PAD, 8), jnp.float32),
        pltpu.SemaphoreType.DMA,
    ],
)
def _deg_kernel(ones_hbm, dst2d_hbm, out_hbm, ones_v, didx, acc_sh, sem):
    c = lax.axis_index("c")
    s = lax.axis_index("s")
    pltpu.sync_copy(ones_hbm, ones_v)
    pltpu.sync_copy(dst2d_hbm.at[pl.ds(s * DEG_CPT, DEG_CPT)], didx)
    rpt = N_PAD // 16
    for r in range(rpt // 128):
        pltpu.sync_copy(ones_v, acc_sh.at[pl.ds(s * rpt + r * 128, 128)])
    plsc.subcore_barrier()

    def body(g, carry):
        ds = [
            pltpu.async_copy(ones_v, acc_sh.at[didx.at[g * 10 + j]], sem, add=True)
            for j in range(10)
        ]
        for d in ds:
            d.wait()
        return carry

    lax.fori_loop(0, DEG_CPT // 10, body, 0)
    plsc.subcore_barrier()
    orow = N_PAD // 32
    ob = c * (N_PAD // 2) + s * orow
    pltpu.sync_copy(acc_sh.at[pl.ds(ob, orow)], out_hbm.at[pl.ds(ob, orow)])


@functools.partial(
    pl.kernel,
    out_type=jax.ShapeDtypeStruct((2 * N_PAD, 128), jnp.float32),
    mesh=_MESH,
    scratch_types=[
        pltpu.VMEM((4, CHUNK), jnp.int32),
        pltpu.VMEM((4, CHUNK), jnp.int32),
        pltpu.VMEM((3, CHUNK, 128), jnp.float32),
        pltpu.VMEM_SHARED((ACC_ROWS, 128), jnp.float32),
        pltpu.SemaphoreType.DMA,
        pltpu.SemaphoreType.DMA,
        pltpu.SemaphoreType.DMA,
    ],
)
def _agg_kernel(
    table_hbm, src1_hbm, dst1_hbm, out_hbm, didxr, sidxr, rows, acc_sh,
    isem, gsem, ssem,
):
    c = lax.axis_index("c")
    s = lax.axis_index("s")
    b = s * ACC_RPT
    eb = c * E_PAD + s * (CPT * CHUNK)
    db = s * (CPT * CHUNK)
    pltpu.sync_copy(
        table_hbm.at[pl.ds(c * N_PAD + b, ACC_RPT)], acc_sh.at[pl.ds(b, ACC_RPT)]
    )
    for r in range(3):
        pltpu.async_copy(
            src1_hbm.at[pl.ds(eb + r * CHUNK, CHUNK)], sidxr.at[r], isem
        )
        pltpu.async_copy(
            dst1_hbm.at[pl.ds(db + r * CHUNK, CHUNK)], didxr.at[r], isem
        )
    plsc.subcore_barrier()
    for r in range(2):
        pltpu.make_async_copy(
            src1_hbm.at[pl.ds(eb, CHUNK)], sidxr.at[r], isem
        ).wait()
        pltpu.make_async_copy(
            dst1_hbm.at[pl.ds(db, CHUNK)], didxr.at[r], isem
        ).wait()
        pltpu.async_copy(table_hbm.at[sidxr.at[r]], rows.at[r], gsem)

    def slot(k, carry):
        pk = lax.rem(k, 3)

        pltpu.make_async_copy(
            table_hbm.at[pl.ds(0, CHUNK)], rows.at[pk], gsem
        ).wait()
        pltpu.async_copy(rows.at[pk], acc_sh.at[didxr.at[lax.rem(k, 4)]], ssem,
                         add=True)

        @pl.when(k >= 1)
        def _():
            pltpu.make_async_copy(
                rows.at[pk], acc_sh.at[didxr.at[lax.rem(k, 4)]], ssem
            ).wait()

        @pl.when(k + 2 < CPT)
        def _():
            pltpu.make_async_copy(
                src1_hbm.at[pl.ds(eb, CHUNK)], sidxr.at[0], isem
            ).wait()
            pltpu.make_async_copy(
                dst1_hbm.at[pl.ds(db, CHUNK)], didxr.at[0], isem
            ).wait()
            pltpu.async_copy(
                table_hbm.at[sidxr.at[lax.rem(k + 2, 4)]],
                rows.at[lax.rem(k + 2, 3)],
                gsem,
            )

        @pl.when(k + 3 < CPT)
        def _():
            pltpu.async_copy(
                src1_hbm.at[pl.ds(eb + (k + 3) * CHUNK, CHUNK)],
                sidxr.at[lax.rem(k + 3, 4)],
                isem,
            )
            pltpu.async_copy(
                dst1_hbm.at[pl.ds(db + (k + 3) * CHUNK, CHUNK)],
                didxr.at[lax.rem(k + 3, 4)],
                isem,
            )

        return carry

    lax.fori_loop(0, CPT, slot, 0)
    pltpu.make_async_copy(
        rows.at[lax.rem(CPT - 1, 3)], acc_sh.at[didxr.at[0]], ssem
    ).wait()
    plsc.subcore_barrier()
    pltpu.sync_copy(
        acc_sh.at[pl.ds(b, ACC_RPT)], out_hbm.at[pl.ds(c * N_PAD + b, ACC_RPT)]
    )


_RB = 1024
_NRB = N_PAD // _RB


def _lin1_body(x_ref, w_ref, deg_ref, tab_ref, dinv_ref):
    di8 = lax.rsqrt(deg_ref[...])
    dinv_ref[...] = di8
    h = jnp.dot(x_ref[...], w_ref[...], preferred_element_type=jnp.float32)
    tab_ref[...] = h * di8[:, :1]


def _lin1(x_pad, w1, deg8):
    return pl.pallas_call(
        _lin1_body,
        grid=(_NRB, 2),
        in_specs=[
            pl.BlockSpec((_RB, D_IN), lambda i, j: (i, 0)),
            pl.BlockSpec((D_IN, 128), lambda i, j: (0, j)),
            pl.BlockSpec((_RB, 8), lambda i, j: (i, 0)),
        ],
        out_specs=[
            pl.BlockSpec((_RB, 128), lambda i, j: (j * _NRB + i, 0)),
            pl.BlockSpec((_RB, 8), lambda i, j: (i, 0)),
        ],
        out_shape=[
            jax.ShapeDtypeStruct((2 * N_PAD, 128), jnp.float32),
            jax.ShapeDtypeStruct((N_PAD, 8), jnp.float32),
        ],
    )(x_pad, w1, deg8)


def _mid_body(agg_ref, dinv_ref, b_ref, w_ref, out_ref):
    di = dinv_ref[:, :1]
    full = jnp.concatenate([agg_ref[0], agg_ref[1]], axis=1)
    h = jnp.maximum(full * di + b_ref[0:1, :], 0.0)
    out_ref[...] = jnp.dot(h, w_ref[...], preferred_element_type=jnp.float32) * di


def _mid(agg, dinv8, b8, w2):
    return pl.pallas_call(
        _mid_body,
        grid=(_NRB, 2),
        in_specs=[
            pl.BlockSpec((2, _RB, 128), lambda i, j: (0, i, 0)),
            pl.BlockSpec((_RB, 8), lambda i, j: (i, 0)),
            pl.BlockSpec((8, D_HID), lambda i, j: (0, 0)),
            pl.BlockSpec((D_HID, 128), lambda i, j: (0, j)),
        ],
        out_specs=pl.BlockSpec((_RB, 128), lambda i, j: (j * _NRB + i, 0)),
        out_shape=jax.ShapeDtypeStruct((2 * N_PAD, 128), jnp.float32),
    )(agg, dinv8, b8, w2)


def _head_body(agg_ref, dinv_ref, b_ref, wr_ref, br_ref, out_ref):
    di = dinv_ref[:, :1]
    full = jnp.concatenate([agg_ref[0], agg_ref[1]], axis=1)
    h = jnp.maximum(full * di + b_ref[0:1, :], 0.0)
    lg = jnp.dot(h, wr_ref[...], preferred_element_type=jnp.float32) + br_ref[0:1, :]
    m = jnp.max(lg, axis=1, keepdims=True)
    e = jnp.exp(lg - m)
    out_ref[...] = e / jnp.sum(e, axis=1, keepdims=True)


def _head(agg, dinv8, b8, wr, br8):
    return pl.pallas_call(
        _head_body,
        grid=(_NRB,),
        in_specs=[
            pl.BlockSpec((2, _RB, 128), lambda i: (0, i, 0)),
            pl.BlockSpec((_RB, 8), lambda i: (i, 0)),
            pl.BlockSpec((8, D_HID), lambda i: (0, 0)),
            pl.BlockSpec((D_HID, D_OUT), lambda i: (0, 0)),
            pl.BlockSpec((8, D_OUT), lambda i: (0, 0)),
        ],
        out_specs=pl.BlockSpec((_RB, D_OUT), lambda i: (i, 0)),
        out_shape=jax.ShapeDtypeStruct((N_PAD, D_OUT), jnp.float32),
    )(agg, dinv8, b8, wr, br8)


@jax.jit
def _run(x, edge_index, w1, b1, w2, b2, wr, br):
    src = edge_index[0]
    dst = edge_index[1]
    pad = E_PAD - E_EDGES
    srcp = jnp.concatenate([src, jnp.zeros((pad,), jnp.int32)])
    dstp = jnp.concatenate([dst, jnp.full((pad,), N_NODES, jnp.int32)])
    src1 = jnp.concatenate([srcp, srcp + N_PAD])
    dst1 = dstp

    dpad = DEG_E_PAD - E_EDGES
    dst2d = jnp.concatenate(
        [dst, jnp.full((dpad,), N_NODES, jnp.int32)]
    ).reshape(DEG_E_ROWS, 128)

    ones8 = jnp.ones((128, 8), jnp.float32)
    deg8 = _deg_kernel(ones8, dst2d)

    x_pad = jnp.concatenate([x, jnp.zeros((N_PAD - N_NODES, D_IN), jnp.float32)])
    tab1, dinv8 = _lin1(x_pad, w1, deg8)

    agg1 = _agg_kernel(tab1, src1, dst1).reshape(2, N_PAD, 128)

    b1_8 = jnp.broadcast_to(b1, (8, D_HID))
    tab2 = _mid(agg1, dinv8, b1_8, w2)

    agg2 = _agg_kernel(tab2, src1, dst1).reshape(2, N_PAD, 128)

    b2_8 = jnp.broadcast_to(b2, (8, D_HID))
    br_8 = jnp.broadcast_to(br, (8, D_OUT))
    out = _head(agg2, dinv8, b2_8, wr, br_8)
    return out[:N_NODES]


def kernel(x, edge_index, W1, b1, W2, b2, Wr, br):
    return _run(x, edge_index, W1, b1, W2, b2, Wr, br)


# speedup vs baseline: 17.5938x; 1.1854x over previous
import functools

import jax
import jax.numpy as jnp
from jax import lax
from jax.experimental import pallas as pl
from jax.experimental.pallas import tpu as pltpu
from jax.experimental.pallas import tpu_sc as plsc

N_NODES = 10000
N_PAD = 10240
E_EDGES = 160000
CHUNK = 88
CPT = 114
E_PAD = 16 * CPT * CHUNK
ACC_ROWS = 10112
ACC_RPT = ACC_ROWS // 16
DEG_CPT = 80
DEG_E_PAD = 16 * DEG_CPT * 128
DEG_E_ROWS = DEG_E_PAD // 128
D_IN = 256
D_HID = 256
D_OUT = 128

_MESH = plsc.VectorSubcoreMesh(
    core_axis_name="c", subcore_axis_name="s", num_cores=2, num_subcores=16
)


@functools.partial(
    pl.kernel,
    out_type=jax.ShapeDtypeStruct((N_PAD, 8), jnp.float32),
    mesh=_MESH,
    scratch_types=[
        pltpu.VMEM((128, 8), jnp.float32),
        pltpu.VMEM((DEG_CPT, 128), jnp.int32),
        pltpu.VMEM_SHARED((N_PAD, 8), jnp.float32),
        pltpu.SemaphoreType.DMA,
    ],
)
def _deg_kernel(ones_hbm, dst2d_hbm, out_hbm, ones_v, didx, acc_sh, sem):
    c = lax.axis_index("c")
    s = lax.axis_index("s")
    pltpu.sync_copy(ones_hbm, ones_v)
    pltpu.sync_copy(dst2d_hbm.at[pl.ds(s * DEG_CPT, DEG_CPT)], didx)
    rpt = N_PAD // 16
    for r in range(rpt // 128):
        pltpu.sync_copy(ones_v, acc_sh.at[pl.ds(s * rpt + r * 128, 128)])
    plsc.subcore_barrier()

    def body(g, carry):
        ds = [
            pltpu.async_copy(ones_v, acc_sh.at[didx.at[g * 10 + j]], sem, add=True)
            for j in range(10)
        ]
        for d in ds:
            d.wait()
        return carry

    lax.fori_loop(0, DEG_CPT // 10, body, 0)
    plsc.subcore_barrier()
    orow = N_PAD // 32
    ob = c * (N_PAD // 2) + s * orow
    pltpu.sync_copy(acc_sh.at[pl.ds(ob, orow)], out_hbm.at[pl.ds(ob, orow)])


@functools.partial(
    pl.kernel,
    out_type=jax.ShapeDtypeStruct((2 * N_PAD, 128), jnp.float32),
    mesh=_MESH,
    scratch_types=[
        pltpu.VMEM((5, CHUNK), jnp.int32),
        pltpu.VMEM((5, CHUNK), jnp.int32),
        pltpu.VMEM((4, CHUNK, 128), jnp.float32),
        pltpu.VMEM_SHARED((ACC_ROWS, 128), jnp.float32),
        pltpu.SemaphoreType.DMA,
        pltpu.SemaphoreType.DMA,
        pltpu.SemaphoreType.DMA,
    ],
)
def _agg_kernel(
    table_hbm, src1_hbm, dst1_hbm, out_hbm, didxr, sidxr, rows, acc_sh,
    isem, gsem, ssem,
):
    c = lax.axis_index("c")
    s = lax.axis_index("s")
    b = s * ACC_RPT
    eb = c * E_PAD + s * (CPT * CHUNK)
    db = s * (CPT * CHUNK)
    pltpu.sync_copy(
        table_hbm.at[pl.ds(c * N_PAD + b, ACC_RPT)], acc_sh.at[pl.ds(b, ACC_RPT)]
    )
    for r in range(4):
        pltpu.async_copy(
            src1_hbm.at[pl.ds(eb + r * CHUNK, CHUNK)], sidxr.at[r], isem
        )
        pltpu.async_copy(
            dst1_hbm.at[pl.ds(db + r * CHUNK, CHUNK)], didxr.at[r], isem
        )
    plsc.subcore_barrier()
    for r in range(3):
        pltpu.make_async_copy(
            src1_hbm.at[pl.ds(eb, CHUNK)], sidxr.at[r], isem
        ).wait()
        pltpu.make_async_copy(
            dst1_hbm.at[pl.ds(db, CHUNK)], didxr.at[r], isem
        ).wait()
        pltpu.async_copy(table_hbm.at[sidxr.at[r]], rows.at[r], gsem)

    def slot(k, carry):
        pk = lax.rem(k, 4)

        pltpu.make_async_copy(
            table_hbm.at[pl.ds(0, CHUNK)], rows.at[pk], gsem
        ).wait()
        pltpu.async_copy(rows.at[pk], acc_sh.at[didxr.at[lax.rem(k, 5)]], ssem,
                         add=True)

        @pl.when(k >= 1)
        def _():
            pltpu.make_async_copy(
                rows.at[pk], acc_sh.at[didxr.at[lax.rem(k, 5)]], ssem
            ).wait()

        @pl.when(k + 3 < CPT)
        def _():
            pltpu.make_async_copy(
                src1_hbm.at[pl.ds(eb, CHUNK)], sidxr.at[0], isem
            ).wait()
            pltpu.make_async_copy(
                dst1_hbm.at[pl.ds(db, CHUNK)], didxr.at[0], isem
            ).wait()
            pltpu.async_copy(
                table_hbm.at[sidxr.at[lax.rem(k + 3, 5)]],
                rows.at[lax.rem(k + 3, 4)],
                gsem,
            )

        @pl.when(k + 4 < CPT)
        def _():
            pltpu.async_copy(
                src1_hbm.at[pl.ds(eb + (k + 4) * CHUNK, CHUNK)],
                sidxr.at[lax.rem(k + 4, 5)],
                isem,
            )
            pltpu.async_copy(
                dst1_hbm.at[pl.ds(db + (k + 4) * CHUNK, CHUNK)],
                didxr.at[lax.rem(k + 4, 5)],
                isem,
            )

        return carry

    lax.fori_loop(0, CPT, slot, 0)
    pltpu.make_async_copy(
        rows.at[lax.rem(CPT - 1, 4)], acc_sh.at[didxr.at[0]], ssem
    ).wait()
    plsc.subcore_barrier()
    pltpu.sync_copy(
        acc_sh.at[pl.ds(b, ACC_RPT)], out_hbm.at[pl.ds(c * N_PAD + b, ACC_RPT)]
    )


_RB = 1024
_NRB = N_PAD // _RB


def _lin1_body(x_ref, w_ref, deg_ref, tab_ref, dinv_ref):
    di8 = lax.rsqrt(deg_ref[...])
    dinv_ref[...] = di8
    h = jnp.dot(x_ref[...], w_ref[...], preferred_element_type=jnp.float32)
    tab_ref[...] = h * di8[:, :1]


def _lin1(x_pad, w1, deg8):
    return pl.pallas_call(
        _lin1_body,
        grid=(_NRB, 2),
        in_specs=[
            pl.BlockSpec((_RB, D_IN), lambda i, j: (i, 0)),
            pl.BlockSpec((D_IN, 128), lambda i, j: (0, j)),
            pl.BlockSpec((_RB, 8), lambda i, j: (i, 0)),
        ],
        out_specs=[
            pl.BlockSpec((_RB, 128), lambda i, j: (j * _NRB + i, 0)),
            pl.BlockSpec((_RB, 8), lambda i, j: (i, 0)),
        ],
        out_shape=[
            jax.ShapeDtypeStruct((2 * N_PAD, 128), jnp.float32),
            jax.ShapeDtypeStruct((N_PAD, 8), jnp.float32),
        ],
    )(x_pad, w1, deg8)


def _mid_body(agg_ref, dinv_ref, b_ref, w_ref, out_ref):
    di = dinv_ref[:, :1]
    full = jnp.concatenate([agg_ref[0], agg_ref[1]], axis=1)
    h = jnp.maximum(full * di + b_ref[0:1, :], 0.0)
    out_ref[...] = jnp.dot(h, w_ref[...], preferred_element_type=jnp.float32) * di


def _mid(agg, dinv8, b8, w2):
    return pl.pallas_call(
        _mid_body,
        grid=(_NRB, 2),
        in_specs=[
            pl.BlockSpec((2, _RB, 128), lambda i, j: (0, i, 0)),
            pl.BlockSpec((_RB, 8), lambda i, j: (i, 0)),
            pl.BlockSpec((8, D_HID), lambda i, j: (0, 0)),
            pl.BlockSpec((D_HID, 128), lambda i, j: (0, j)),
        ],
        out_specs=pl.BlockSpec((_RB, 128), lambda i, j: (j * _NRB + i, 0)),
        out_shape=jax.ShapeDtypeStruct((2 * N_PAD, 128), jnp.float32),
    )(agg, dinv8, b8, w2)


def _head_body(agg_ref, dinv_ref, b_ref, wr_ref, br_ref, out_ref):
    di = dinv_ref[:, :1]
    full = jnp.concatenate([agg_ref[0], agg_ref[1]], axis=1)
    h = jnp.maximum(full * di + b_ref[0:1, :], 0.0)
    lg = jnp.dot(h, wr_ref[...], preferred_element_type=jnp.float32) + br_ref[0:1, :]
    m = jnp.max(lg, axis=1, keepdims=True)
    e = jnp.exp(lg - m)
    out_ref[...] = e / jnp.sum(e, axis=1, keepdims=True)


def _head(agg, dinv8, b8, wr, br8):
    return pl.pallas_call(
        _head_body,
        grid=(_NRB,),
        in_specs=[
            pl.BlockSpec((2, _RB, 128), lambda i: (0, i, 0)),
            pl.BlockSpec((_RB, 8), lambda i: (i, 0)),
            pl.BlockSpec((8, D_HID), lambda i: (0, 0)),
            pl.BlockSpec((D_HID, D_OUT), lambda i: (0, 0)),
            pl.BlockSpec((8, D_OUT), lambda i: (0, 0)),
        ],
        out_specs=pl.BlockSpec((_RB, D_OUT), lambda i: (i, 0)),
        out_shape=jax.ShapeDtypeStruct((N_PAD, D_OUT), jnp.float32),
    )(agg, dinv8, b8, wr, br8)


@jax.jit
def _run(x, edge_index, w1, b1, w2, b2, wr, br):
    src = edge_index[0]
    dst = edge_index[1]
    pad = E_PAD - E_EDGES
    srcp = jnp.concatenate([src, jnp.zeros((pad,), jnp.int32)])
    dstp = jnp.concatenate([dst, jnp.full((pad,), N_NODES, jnp.int32)])
    src1 = jnp.concatenate([srcp, srcp + N_PAD])
    dst1 = dstp

    dpad = DEG_E_PAD - E_EDGES
    dst2d = jnp.concatenate(
        [dst, jnp.full((dpad,), N_NODES, jnp.int32)]
    ).reshape(DEG_E_ROWS, 128)

    ones8 = jnp.ones((128, 8), jnp.float32)
    deg8 = _deg_kernel(ones8, dst2d)

    x_pad = jnp.concatenate([x, jnp.zeros((N_PAD - N_NODES, D_IN), jnp.float32)])
    tab1, dinv8 = _lin1(x_pad, w1, deg8)

    agg1 = _agg_kernel(tab1, src1, dst1).reshape(2, N_PAD, 128)

    b1_8 = jnp.broadcast_to(b1, (8, D_HID))
    tab2 = _mid(agg1, dinv8, b1_8, w2)

    agg2 = _agg_kernel(tab2, src1, dst1).reshape(2, N_PAD, 128)

    b2_8 = jnp.broadcast_to(b2, (8, D_HID))
    br_8 = jnp.broadcast_to(br, (8, D_OUT))
    out = _head(agg2, dinv8, b2_8, wr, br_8)
    return out[:N_NODES]


def kernel(x, edge_index, W1, b1, W2, b2, Wr, br):
    return _run(x, edge_index, W1, b1, W2, b2, Wr, br)


# speedup vs baseline: 19.2239x; 1.0926x over previous
import functools

import jax
import jax.numpy as jnp
from jax import lax
from jax.experimental import pallas as pl
from jax.experimental.pallas import tpu as pltpu
from jax.experimental.pallas import tpu_sc as plsc

N_NODES = 10000
N_PAD = 10240
E_EDGES = 160000
CHUNK = 72
CPT = 139
E_PAD = 16 * CPT * CHUNK
ACC_ROWS = 10112
ACC_RPT = ACC_ROWS // 16
DEG_CPT = 80
DEG_E_PAD = 16 * DEG_CPT * 128
DEG_E_ROWS = DEG_E_PAD // 128
D_IN = 256
D_HID = 256
D_OUT = 128

_MESH = plsc.VectorSubcoreMesh(
    core_axis_name="c", subcore_axis_name="s", num_cores=2, num_subcores=16
)


@functools.partial(
    pl.kernel,
    out_type=jax.ShapeDtypeStruct((N_PAD, 8), jnp.float32),
    mesh=_MESH,
    scratch_types=[
        pltpu.VMEM((128, 8), jnp.float32),
        pltpu.VMEM((DEG_CPT, 128), jnp.int32),
        pltpu.VMEM_SHARED((N_PAD, 8), jnp.float32),
        pltpu.SemaphoreType.DMA,
    ],
)
def _deg_kernel(ones_hbm, dst2d_hbm, out_hbm, ones_v, didx, acc_sh, sem):
    c = lax.axis_index("c")
    s = lax.axis_index("s")
    pltpu.sync_copy(ones_hbm, ones_v)
    pltpu.sync_copy(dst2d_hbm.at[pl.ds(s * DEG_CPT, DEG_CPT)], didx)
    rpt = N_PAD // 16
    for r in range(rpt // 128):
        pltpu.sync_copy(ones_v, acc_sh.at[pl.ds(s * rpt + r * 128, 128)])
    plsc.subcore_barrier()

    def body(g, carry):
        ds = [
            pltpu.async_copy(ones_v, acc_sh.at[didx.at[g * 10 + j]], sem, add=True)
            for j in range(10)
        ]
        for d in ds:
            d.wait()
        return carry

    lax.fori_loop(0, DEG_CPT // 10, body, 0)
    plsc.subcore_barrier()
    orow = N_PAD // 32
    ob = c * (N_PAD // 2) + s * orow
    pltpu.sync_copy(acc_sh.at[pl.ds(ob, orow)], out_hbm.at[pl.ds(ob, orow)])


@functools.partial(
    pl.kernel,
    out_type=jax.ShapeDtypeStruct((2 * N_PAD, 128), jnp.float32),
    mesh=_MESH,
    scratch_types=[
        pltpu.VMEM((6, CHUNK), jnp.int32),
        pltpu.VMEM((6, CHUNK), jnp.int32),
        pltpu.VMEM((5, CHUNK, 128), jnp.float32),
        pltpu.VMEM_SHARED((ACC_ROWS, 128), jnp.float32),
        pltpu.SemaphoreType.DMA,
        pltpu.SemaphoreType.DMA,
        pltpu.SemaphoreType.DMA,
    ],
)
def _agg_kernel(
    table_hbm, src1_hbm, dst1_hbm, out_hbm, didxr, sidxr, rows, acc_sh,
    isem, gsem, ssem,
):
    c = lax.axis_index("c")
    s = lax.axis_index("s")
    b = s * ACC_RPT
    eb = c * E_PAD + s * (CPT * CHUNK)
    db = s * (CPT * CHUNK)
    pltpu.sync_copy(
        table_hbm.at[pl.ds(c * N_PAD + b, ACC_RPT)], acc_sh.at[pl.ds(b, ACC_RPT)]
    )
    for r in range(5):
        pltpu.async_copy(
            src1_hbm.at[pl.ds(eb + r * CHUNK, CHUNK)], sidxr.at[r], isem
        )
        pltpu.async_copy(
            dst1_hbm.at[pl.ds(db + r * CHUNK, CHUNK)], didxr.at[r], isem
        )
    plsc.subcore_barrier()
    for r in range(4):
        pltpu.make_async_copy(
            src1_hbm.at[pl.ds(eb, CHUNK)], sidxr.at[r], isem
        ).wait()
        pltpu.make_async_copy(
            dst1_hbm.at[pl.ds(db, CHUNK)], didxr.at[r], isem
        ).wait()
        pltpu.async_copy(table_hbm.at[sidxr.at[r]], rows.at[r], gsem)

    def slot(k, carry):
        pk = lax.rem(k, 5)

        pltpu.make_async_copy(
            table_hbm.at[pl.ds(0, CHUNK)], rows.at[pk], gsem
        ).wait()
        pltpu.async_copy(rows.at[pk], acc_sh.at[didxr.at[lax.rem(k, 6)]], ssem,
                         add=True)

        @pl.when(k >= 1)
        def _():
            pltpu.make_async_copy(
                rows.at[pk], acc_sh.at[didxr.at[lax.rem(k, 6)]], ssem
            ).wait()

        @pl.when(k + 4 < CPT)
        def _():
            pltpu.make_async_copy(
                src1_hbm.at[pl.ds(eb, CHUNK)], sidxr.at[0], isem
            ).wait()
            pltpu.make_async_copy(
                dst1_hbm.at[pl.ds(db, CHUNK)], didxr.at[0], isem
            ).wait()
            pltpu.async_copy(
                table_hbm.at[sidxr.at[lax.rem(k + 4, 6)]],
                rows.at[lax.rem(k + 4, 5)],
                gsem,
            )

        @pl.when(k + 5 < CPT)
        def _():
            pltpu.async_copy(
                src1_hbm.at[pl.ds(eb + (k + 5) * CHUNK, CHUNK)],
                sidxr.at[lax.rem(k + 5, 6)],
                isem,
            )
            pltpu.async_copy(
                dst1_hbm.at[pl.ds(db + (k + 5) * CHUNK, CHUNK)],
                didxr.at[lax.rem(k + 5, 6)],
                isem,
            )

        return carry

    lax.fori_loop(0, CPT, slot, 0)
    pltpu.make_async_copy(
        rows.at[lax.rem(CPT - 1, 5)], acc_sh.at[didxr.at[0]], ssem
    ).wait()
    plsc.subcore_barrier()
    pltpu.sync_copy(
        acc_sh.at[pl.ds(b, ACC_RPT)], out_hbm.at[pl.ds(c * N_PAD + b, ACC_RPT)]
    )


_RB = 1024
_NRB = N_PAD // _RB


def _lin1_body(x_ref, w_ref, deg_ref, tab_ref, dinv_ref):
    di8 = lax.rsqrt(deg_ref[...])
    dinv_ref[...] = di8
    h = jnp.dot(x_ref[...], w_ref[...], preferred_element_type=jnp.float32)
    tab_ref[...] = h * di8[:, :1]


def _lin1(x_pad, w1, deg8):
    return pl.pallas_call(
        _lin1_body,
        grid=(_NRB, 2),
        in_specs=[
            pl.BlockSpec((_RB, D_IN), lambda i, j: (i, 0)),
            pl.BlockSpec((D_IN, 128), lambda i, j: (0, j)),
            pl.BlockSpec((_RB, 8), lambda i, j: (i, 0)),
        ],
        out_specs=[
            pl.BlockSpec((_RB, 128), lambda i, j: (j * _NRB + i, 0)),
            pl.BlockSpec((_RB, 8), lambda i, j: (i, 0)),
        ],
        out_shape=[
            jax.ShapeDtypeStruct((2 * N_PAD, 128), jnp.float32),
            jax.ShapeDtypeStruct((N_PAD, 8), jnp.float32),
        ],
    )(x_pad, w1, deg8)


def _mid_body(agg_ref, dinv_ref, b_ref, w_ref, out_ref):
    di = dinv_ref[:, :1]
    full = jnp.concatenate([agg_ref[0], agg_ref[1]], axis=1)
    h = jnp.maximum(full * di + b_ref[0:1, :], 0.0)
    out_ref[...] = jnp.dot(h, w_ref[...], preferred_element_type=jnp.float32) * di


def _mid(agg, dinv8, b8, w2):
    return pl.pallas_call(
        _mid_body,
        grid=(_NRB, 2),
        in_specs=[
            pl.BlockSpec((2, _RB, 128), lambda i, j: (0, i, 0)),
            pl.BlockSpec((_RB, 8), lambda i, j: (i, 0)),
            pl.BlockSpec((8, D_HID), lambda i, j: (0, 0)),
            pl.BlockSpec((D_HID, 128), lambda i, j: (0, j)),
        ],
        out_specs=pl.BlockSpec((_RB, 128), lambda i, j: (j * _NRB + i, 0)),
        out_shape=jax.ShapeDtypeStruct((2 * N_PAD, 128), jnp.float32),
    )(agg, dinv8, b8, w2)


def _head_body(agg_ref, dinv_ref, b_ref, wr_ref, br_ref, out_ref):
    di = dinv_ref[:, :1]
    full = jnp.concatenate([agg_ref[0], agg_ref[1]], axis=1)
    h = jnp.maximum(full * di + b_ref[0:1, :], 0.0)
    lg = jnp.dot(h, wr_ref[...], preferred_element_type=jnp.float32) + br_ref[0:1, :]
    m = jnp.max(lg, axis=1, keepdims=True)
    e = jnp.exp(lg - m)
    out_ref[...] = e / jnp.sum(e, axis=1, keepdims=True)


def _head(agg, dinv8, b8, wr, br8):
    return pl.pallas_call(
        _head_body,
        grid=(_NRB,),
        in_specs=[
            pl.BlockSpec((2, _RB, 128), lambda i: (0, i, 0)),
            pl.BlockSpec((_RB, 8), lambda i: (i, 0)),
            pl.BlockSpec((8, D_HID), lambda i: (0, 0)),
            pl.BlockSpec((D_HID, D_OUT), lambda i: (0, 0)),
            pl.BlockSpec((8, D_OUT), lambda i: (0, 0)),
        ],
        out_specs=pl.BlockSpec((_RB, D_OUT), lambda i: (i, 0)),
        out_shape=jax.ShapeDtypeStruct((N_PAD, D_OUT), jnp.float32),
    )(agg, dinv8, b8, wr, br8)


@jax.jit
def _run(x, edge_index, w1, b1, w2, b2, wr, br):
    src = edge_index[0]
    dst = edge_index[1]
    pad = E_PAD - E_EDGES
    srcp = jnp.concatenate([src, jnp.zeros((pad,), jnp.int32)])
    dstp = jnp.concatenate([dst, jnp.full((pad,), N_NODES, jnp.int32)])
    src1 = jnp.concatenate([srcp, srcp + N_PAD])
    dst1 = dstp

    dpad = DEG_E_PAD - E_EDGES
    dst2d = jnp.concatenate(
        [dst, jnp.full((dpad,), N_NODES, jnp.int32)]
    ).reshape(DEG_E_ROWS, 128)

    ones8 = jnp.ones((128, 8), jnp.float32)
    deg8 = _deg_kernel(ones8, dst2d)

    x_pad = jnp.concatenate([x, jnp.zeros((N_PAD - N_NODES, D_IN), jnp.float32)])
    tab1, dinv8 = _lin1(x_pad, w1, deg8)

    agg1 = _agg_kernel(tab1, src1, dst1).reshape(2, N_PAD, 128)

    b1_8 = jnp.broadcast_to(b1, (8, D_HID))
    tab2 = _mid(agg1, dinv8, b1_8, w2)

    agg2 = _agg_kernel(tab2, src1, dst1).reshape(2, N_PAD, 128)

    b2_8 = jnp.broadcast_to(b2, (8, D_HID))
    br_8 = jnp.broadcast_to(br, (8, D_OUT))
    out = _head(agg2, dinv8, b2_8, wr, br_8)
    return out[:N_NODES]


def kernel(x, edge_index, W1, b1, W2, b2, Wr, br):
    return _run(x, edge_index, W1, b1, W2, b2, Wr, br)


# speedup vs baseline: 20.6297x; 1.0731x over previous
import functools

import jax
import jax.numpy as jnp
from jax import lax
from jax.experimental import pallas as pl
from jax.experimental.pallas import tpu as pltpu
from jax.experimental.pallas import tpu_sc as plsc

N_NODES = 10000
N_PAD = 10240
E_EDGES = 160000
CHUNK = 72
CPT = 139
E_PAD = 16 * CPT * CHUNK
ACC_ROWS = 10112
ACC_RPT = ACC_ROWS // 16
DEG_CPT = 80
DEG_E_PAD = 16 * DEG_CPT * 128
DEG_E_ROWS = DEG_E_PAD // 128
D_IN = 256
D_HID = 256
D_OUT = 128

_MESH = plsc.VectorSubcoreMesh(
    core_axis_name="c", subcore_axis_name="s", num_cores=2, num_subcores=16
)


@functools.partial(
    pl.kernel,
    out_type=jax.ShapeDtypeStruct((N_PAD, 8), jnp.float32),
    mesh=_MESH,
    scratch_types=[
        pltpu.VMEM((128, 8), jnp.float32),
        pltpu.VMEM((DEG_CPT, 128), jnp.int32),
        pltpu.VMEM_SHARED((N_PAD, 8), jnp.float32),
        pltpu.SemaphoreType.DMA,
    ],
)
def _deg_kernel(ones_hbm, dst2d_hbm, out_hbm, ones_v, didx, acc_sh, sem):
    c = lax.axis_index("c")
    s = lax.axis_index("s")
    pltpu.sync_copy(ones_hbm, ones_v)
    pltpu.sync_copy(dst2d_hbm.at[pl.ds(s * DEG_CPT, DEG_CPT)], didx)
    rpt = N_PAD // 16
    for r in range(rpt // 128):
        pltpu.sync_copy(ones_v, acc_sh.at[pl.ds(s * rpt + r * 128, 128)])
    plsc.subcore_barrier()

    def body(k, carry):
        pltpu.async_copy(ones_v, acc_sh.at[didx.at[k]], sem, add=True)

        @pl.when(k >= 8)
        def _():
            pltpu.make_async_copy(ones_v, acc_sh.at[didx.at[0]], sem).wait()

        return carry

    lax.fori_loop(0, DEG_CPT, body, 0)
    for _ in range(8):
        pltpu.make_async_copy(ones_v, acc_sh.at[didx.at[0]], sem).wait()
    plsc.subcore_barrier()
    orow = N_PAD // 32
    ob = c * (N_PAD // 2) + s * orow
    pltpu.sync_copy(acc_sh.at[pl.ds(ob, orow)], out_hbm.at[pl.ds(ob, orow)])


@functools.partial(
    pl.kernel,
    out_type=jax.ShapeDtypeStruct((2 * N_PAD, 128), jnp.float32),
    mesh=_MESH,
    scratch_types=[
        pltpu.VMEM((6, CHUNK), jnp.int32),
        pltpu.VMEM((6, CHUNK), jnp.int32),
        pltpu.VMEM((5, CHUNK, 128), jnp.float32),
        pltpu.VMEM_SHARED((ACC_ROWS, 128), jnp.float32),
        pltpu.SemaphoreType.DMA,
        pltpu.SemaphoreType.DMA,
        pltpu.SemaphoreType.DMA,
    ],
)
def _agg_kernel(
    table_hbm, src1_hbm, dst1_hbm, out_hbm, didxr, sidxr, rows, acc_sh,
    isem, gsem, ssem,
):
    c = lax.axis_index("c")
    s = lax.axis_index("s")
    b = s * ACC_RPT
    eb = c * E_PAD + s * (CPT * CHUNK)
    db = s * (CPT * CHUNK)
    pltpu.sync_copy(
        table_hbm.at[pl.ds(c * N_PAD + b, ACC_RPT)], acc_sh.at[pl.ds(b, ACC_RPT)]
    )
    for r in range(5):
        pltpu.async_copy(
            src1_hbm.at[pl.ds(eb + r * CHUNK, CHUNK)], sidxr.at[r], isem
        )
        pltpu.async_copy(
            dst1_hbm.at[pl.ds(db + r * CHUNK, CHUNK)], didxr.at[r], isem
        )
    plsc.subcore_barrier()
    for r in range(4):
        pltpu.make_async_copy(
            src1_hbm.at[pl.ds(eb, CHUNK)], sidxr.at[r], isem
        ).wait()
        pltpu.make_async_copy(
            dst1_hbm.at[pl.ds(db, CHUNK)], didxr.at[r], isem
        ).wait()
        pltpu.async_copy(table_hbm.at[sidxr.at[r]], rows.at[r], gsem)

    def slot(k, carry):
        pk = lax.rem(k, 5)

        pltpu.make_async_copy(
            table_hbm.at[pl.ds(0, CHUNK)], rows.at[pk], gsem
        ).wait()
        pltpu.async_copy(rows.at[pk], acc_sh.at[didxr.at[lax.rem(k, 6)]], ssem,
                         add=True)

        @pl.when(k >= 1)
        def _():
            pltpu.make_async_copy(
                rows.at[pk], acc_sh.at[didxr.at[lax.rem(k, 6)]], ssem
            ).wait()

        @pl.when(k + 4 < CPT)
        def _():
            pltpu.make_async_copy(
                src1_hbm.at[pl.ds(eb, CHUNK)], sidxr.at[0], isem
            ).wait()
            pltpu.make_async_copy(
                dst1_hbm.at[pl.ds(db, CHUNK)], didxr.at[0], isem
            ).wait()
            pltpu.async_copy(
                table_hbm.at[sidxr.at[lax.rem(k + 4, 6)]],
                rows.at[lax.rem(k + 4, 5)],
                gsem,
            )

        @pl.when(k + 5 < CPT)
        def _():
            pltpu.async_copy(
                src1_hbm.at[pl.ds(eb + (k + 5) * CHUNK, CHUNK)],
                sidxr.at[lax.rem(k + 5, 6)],
                isem,
            )
            pltpu.async_copy(
                dst1_hbm.at[pl.ds(db + (k + 5) * CHUNK, CHUNK)],
                didxr.at[lax.rem(k + 5, 6)],
                isem,
            )

        return carry

    lax.fori_loop(0, CPT, slot, 0)
    pltpu.make_async_copy(
        rows.at[lax.rem(CPT - 1, 5)], acc_sh.at[didxr.at[0]], ssem
    ).wait()
    plsc.subcore_barrier()
    pltpu.sync_copy(
        acc_sh.at[pl.ds(b, ACC_RPT)], out_hbm.at[pl.ds(c * N_PAD + b, ACC_RPT)]
    )


_RB = 1024
_NRB = N_PAD // _RB


def _lin1_body(x_ref, w_ref, deg_ref, tab_ref, dinv_ref):
    di8 = lax.rsqrt(deg_ref[...])
    dinv_ref[...] = di8
    h = jnp.dot(x_ref[...], w_ref[...], preferred_element_type=jnp.float32)
    h = h * di8[:, :1]
    tab_ref[0] = h[:, :128]
    tab_ref[1] = h[:, 128:]


def _lin1(x_pad, w1, deg8):
    return pl.pallas_call(
        _lin1_body,
        grid=(_NRB,),
        in_specs=[
            pl.BlockSpec((_RB, D_IN), lambda i: (i, 0)),
            pl.BlockSpec((D_IN, D_HID), lambda i: (0, 0)),
            pl.BlockSpec((_RB, 8), lambda i: (i, 0)),
        ],
        out_specs=[
            pl.BlockSpec((2, _RB, 128), lambda i: (0, i, 0)),
            pl.BlockSpec((_RB, 8), lambda i: (i, 0)),
        ],
        out_shape=[
            jax.ShapeDtypeStruct((2, N_PAD, 128), jnp.float32),
            jax.ShapeDtypeStruct((N_PAD, 8), jnp.float32),
        ],
    )(x_pad, w1, deg8)


def _mid_body(agg_ref, dinv_ref, b_ref, w_ref, out_ref):
    di = dinv_ref[:, :1]
    full = jnp.concatenate([agg_ref[0], agg_ref[1]], axis=1)
    h = jnp.maximum(full * di + b_ref[0:1, :], 0.0)
    o = jnp.dot(h, w_ref[...], preferred_element_type=jnp.float32) * di
    out_ref[0] = o[:, :128]
    out_ref[1] = o[:, 128:]


def _mid(agg, dinv8, b8, w2):
    return pl.pallas_call(
        _mid_body,
        grid=(_NRB,),
        in_specs=[
            pl.BlockSpec((2, _RB, 128), lambda i: (0, i, 0)),
            pl.BlockSpec((_RB, 8), lambda i: (i, 0)),
            pl.BlockSpec((8, D_HID), lambda i: (0, 0)),
            pl.BlockSpec((D_HID, D_HID), lambda i: (0, 0)),
        ],
        out_specs=pl.BlockSpec((2, _RB, 128), lambda i: (0, i, 0)),
        out_shape=jax.ShapeDtypeStruct((2, N_PAD, 128), jnp.float32),
    )(agg, dinv8, b8, w2)


def _head_body(agg_ref, dinv_ref, b_ref, wr_ref, br_ref, out_ref):
    di = dinv_ref[:, :1]
    full = jnp.concatenate([agg_ref[0], agg_ref[1]], axis=1)
    h = jnp.maximum(full * di + b_ref[0:1, :], 0.0)
    lg = jnp.dot(h, wr_ref[...], preferred_element_type=jnp.float32) + br_ref[0:1, :]
    m = jnp.max(lg, axis=1, keepdims=True)
    e = jnp.exp(lg - m)
    out_ref[...] = e / jnp.sum(e, axis=1, keepdims=True)


def _head(agg, dinv8, b8, wr, br8):
    return pl.pallas_call(
        _head_body,
        grid=(_NRB,),
        in_specs=[
            pl.BlockSpec((2, _RB, 128), lambda i: (0, i, 0)),
            pl.BlockSpec((_RB, 8), lambda i: (i, 0)),
            pl.BlockSpec((8, D_HID), lambda i: (0, 0)),
            pl.BlockSpec((D_HID, D_OUT), lambda i: (0, 0)),
            pl.BlockSpec((8, D_OUT), lambda i: (0, 0)),
        ],
        out_specs=pl.BlockSpec((_RB, D_OUT), lambda i: (i, 0)),
        out_shape=jax.ShapeDtypeStruct((N_PAD, D_OUT), jnp.float32),
    )(agg, dinv8, b8, wr, br8)


@jax.jit
def _run(x, edge_index, w1, b1, w2, b2, wr, br):
    src = edge_index[0]
    dst = edge_index[1]
    pad = E_PAD - E_EDGES
    srcp = jnp.concatenate([src, jnp.zeros((pad,), jnp.int32)])
    dstp = jnp.concatenate([dst, jnp.full((pad,), N_NODES, jnp.int32)])
    src1 = jnp.concatenate([srcp, srcp + N_PAD])
    dst1 = dstp

    dpad = DEG_E_PAD - E_EDGES
    dst2d = jnp.concatenate(
        [dst, jnp.full((dpad,), N_NODES, jnp.int32)]
    ).reshape(DEG_E_ROWS, 128)

    ones8 = jnp.ones((128, 8), jnp.float32)
    deg8 = _deg_kernel(ones8, dst2d)

    x_pad = jnp.concatenate([x, jnp.zeros((N_PAD - N_NODES, D_IN), jnp.float32)])
    tab1, dinv8 = _lin1(x_pad, w1, deg8)

    agg1 = _agg_kernel(tab1.reshape(2 * N_PAD, 128), src1, dst1).reshape(
        2, N_PAD, 128
    )

    b1_8 = jnp.broadcast_to(b1, (8, D_HID))
    tab2 = _mid(agg1, dinv8, b1_8, w2)

    agg2 = _agg_kernel(tab2.reshape(2 * N_PAD, 128), src1, dst1).reshape(
        2, N_PAD, 128
    )

    b2_8 = jnp.broadcast_to(b2, (8, D_HID))
    br_8 = jnp.broadcast_to(br, (8, D_OUT))
    out = _head(agg2, dinv8, b2_8, wr, br_8)
    return out[:N_NODES]


def kernel(x, edge_index, W1, b1, W2, b2, Wr, br):
    return _run(x, edge_index, W1, b1, W2, b2, Wr, br)


# speedup vs baseline: 20.9258x; 1.0144x over previous
import functools

import jax
import jax.numpy as jnp
from jax import lax
from jax.experimental import pallas as pl
from jax.experimental.pallas import tpu as pltpu
from jax.experimental.pallas import tpu_sc as plsc

N_NODES = 10000
N_PAD = 10240
E_EDGES = 160000
CHUNK = 72
CPT = 139
E_PAD = 16 * CPT * CHUNK
ACC_ROWS = 10112
ACC_RPT = ACC_ROWS // 16
DEG_CPT = 80
DEG_E_PAD = 16 * DEG_CPT * 128
DEG_E_ROWS = DEG_E_PAD // 128
D_IN = 256
D_HID = 256
D_OUT = 128

_MESH = plsc.VectorSubcoreMesh(
    core_axis_name="c", subcore_axis_name="s", num_cores=2, num_subcores=16
)


@functools.partial(
    pl.kernel,
    out_type=jax.ShapeDtypeStruct((N_PAD, 8), jnp.float32),
    mesh=_MESH,
    scratch_types=[
        pltpu.VMEM((128, 8), jnp.float32),
        pltpu.VMEM((DEG_CPT, 128), jnp.int32),
        pltpu.VMEM_SHARED((N_PAD, 8), jnp.float32),
        pltpu.SemaphoreType.DMA,
    ],
)
def _deg_kernel(ones_hbm, dst2d_hbm, out_hbm, ones_v, didx, acc_sh, sem):
    c = lax.axis_index("c")
    s = lax.axis_index("s")
    pltpu.sync_copy(ones_hbm, ones_v)
    pltpu.sync_copy(dst2d_hbm.at[pl.ds(s * DEG_CPT, DEG_CPT)], didx)
    rpt = N_PAD // 16
    for r in range(rpt // 128):
        pltpu.sync_copy(ones_v, acc_sh.at[pl.ds(s * rpt + r * 128, 128)])
    plsc.subcore_barrier()

    def body(k, carry):
        pltpu.async_copy(ones_v, acc_sh.at[didx.at[k]], sem, add=True)

        @pl.when(k >= 8)
        def _():
            pltpu.make_async_copy(ones_v, acc_sh.at[didx.at[0]], sem).wait()

        return carry

    lax.fori_loop(0, DEG_CPT, body, 0)
    for _ in range(8):
        pltpu.make_async_copy(ones_v, acc_sh.at[didx.at[0]], sem).wait()
    plsc.subcore_barrier()
    orow = N_PAD // 32
    ob = c * (N_PAD // 2) + s * orow
    pltpu.sync_copy(acc_sh.at[pl.ds(ob, orow)], out_hbm.at[pl.ds(ob, orow)])


@functools.partial(
    pl.kernel,
    out_type=jax.ShapeDtypeStruct((2 * N_PAD, 128), jnp.float32),
    mesh=_MESH,
    scratch_types=[
        pltpu.VMEM((6, CHUNK), jnp.int32),
        pltpu.VMEM((6, CHUNK), jnp.int32),
        pltpu.VMEM((5, CHUNK, 128), jnp.float32),
        pltpu.VMEM_SHARED((ACC_ROWS, 128), jnp.float32),
        pltpu.SemaphoreType.DMA,
        pltpu.SemaphoreType.DMA,
        pltpu.SemaphoreType.DMA,
        pltpu.SemaphoreType.DMA,
    ],
)
def _agg_kernel(
    table_hbm, src1_hbm, dst1_hbm, out_hbm, didxr, sidxr, rows, acc_sh,
    isem, gsem, ssem, nsem,
):
    c = lax.axis_index("c")
    s = lax.axis_index("s")
    b = s * ACC_RPT
    eb = c * E_PAD + s * (CPT * CHUNK)
    db = s * (CPT * CHUNK)
    ini = pltpu.async_copy(
        table_hbm.at[pl.ds(c * N_PAD + b, ACC_RPT)], acc_sh.at[pl.ds(b, ACC_RPT)],
        nsem,
    )
    for r in range(5):
        pltpu.async_copy(
            src1_hbm.at[pl.ds(eb + r * CHUNK, CHUNK)], sidxr.at[r], isem
        )
        pltpu.async_copy(
            dst1_hbm.at[pl.ds(db + r * CHUNK, CHUNK)], didxr.at[r], isem
        )
    for r in range(4):
        pltpu.make_async_copy(
            src1_hbm.at[pl.ds(eb, CHUNK)], sidxr.at[r], isem
        ).wait()
        pltpu.make_async_copy(
            dst1_hbm.at[pl.ds(db, CHUNK)], didxr.at[r], isem
        ).wait()
        pltpu.async_copy(table_hbm.at[sidxr.at[r]], rows.at[r], gsem)
    ini.wait()
    plsc.subcore_barrier()

    def slot(k, carry):
        pk = lax.rem(k, 5)

        pltpu.make_async_copy(
            table_hbm.at[pl.ds(0, CHUNK)], rows.at[pk], gsem
        ).wait()
        pltpu.async_copy(rows.at[pk], acc_sh.at[didxr.at[lax.rem(k, 6)]], ssem,
                         add=True)

        @pl.when(k >= 1)
        def _():
            pltpu.make_async_copy(
                rows.at[pk], acc_sh.at[didxr.at[lax.rem(k, 6)]], ssem
            ).wait()

        @pl.when(k + 4 < CPT)
        def _():
            pltpu.make_async_copy(
                src1_hbm.at[pl.ds(eb, CHUNK)], sidxr.at[0], isem
            ).wait()
            pltpu.make_async_copy(
                dst1_hbm.at[pl.ds(db, CHUNK)], didxr.at[0], isem
            ).wait()
            pltpu.async_copy(
                table_hbm.at[sidxr.at[lax.rem(k + 4, 6)]],
                rows.at[lax.rem(k + 4, 5)],
                gsem,
            )

        @pl.when(k + 5 < CPT)
        def _():
            pltpu.async_copy(
                src1_hbm.at[pl.ds(eb + (k + 5) * CHUNK, CHUNK)],
                sidxr.at[lax.rem(k + 5, 6)],
                isem,
            )
            pltpu.async_copy(
                dst1_hbm.at[pl.ds(db + (k + 5) * CHUNK, CHUNK)],
                didxr.at[lax.rem(k + 5, 6)],
                isem,
            )

        return carry

    lax.fori_loop(0, CPT, slot, 0)
    pltpu.make_async_copy(
        rows.at[lax.rem(CPT - 1, 5)], acc_sh.at[didxr.at[0]], ssem
    ).wait()
    plsc.subcore_barrier()
    pltpu.sync_copy(
        acc_sh.at[pl.ds(b, ACC_RPT)], out_hbm.at[pl.ds(c * N_PAD + b, ACC_RPT)]
    )


_RB = 1024
_NRB = N_PAD // _RB


def _lin1_body(x_ref, w_ref, deg_ref, tab_ref, dinv_ref):
    di8 = lax.rsqrt(deg_ref[...])
    dinv_ref[...] = di8
    h = jnp.dot(x_ref[...], w_ref[...], preferred_element_type=jnp.float32)
    h = h * di8[:, :1]
    tab_ref[0] = h[:, :128]
    tab_ref[1] = h[:, 128:]


def _lin1(x_pad, w1, deg8):
    return pl.pallas_call(
        _lin1_body,
        grid=(_NRB,),
        in_specs=[
            pl.BlockSpec((_RB, D_IN), lambda i: (i, 0)),
            pl.BlockSpec((D_IN, D_HID), lambda i: (0, 0)),
            pl.BlockSpec((_RB, 8), lambda i: (i, 0)),
        ],
        out_specs=[
            pl.BlockSpec((2, _RB, 128), lambda i: (0, i, 0)),
            pl.BlockSpec((_RB, 8), lambda i: (i, 0)),
        ],
        out_shape=[
            jax.ShapeDtypeStruct((2, N_PAD, 128), jnp.float32),
            jax.ShapeDtypeStruct((N_PAD, 8), jnp.float32),
        ],
    )(x_pad, w1, deg8)


def _mid_body(agg_ref, dinv_ref, b_ref, w_ref, out_ref):
    di = dinv_ref[:, :1]
    full = jnp.concatenate([agg_ref[0], agg_ref[1]], axis=1)
    h = jnp.maximum(full * di + b_ref[0:1, :], 0.0)
    o = jnp.dot(h, w_ref[...], preferred_element_type=jnp.float32) * di
    out_ref[0] = o[:, :128]
    out_ref[1] = o[:, 128:]


def _mid(agg, dinv8, b8, w2):
    return pl.pallas_call(
        _mid_body,
        grid=(_NRB,),
        in_specs=[
            pl.BlockSpec((2, _RB, 128), lambda i: (0, i, 0)),
            pl.BlockSpec((_RB, 8), lambda i: (i, 0)),
            pl.BlockSpec((8, D_HID), lambda i: (0, 0)),
            pl.BlockSpec((D_HID, D_HID), lambda i: (0, 0)),
        ],
        out_specs=pl.BlockSpec((2, _RB, 128), lambda i: (0, i, 0)),
        out_shape=jax.ShapeDtypeStruct((2, N_PAD, 128), jnp.float32),
    )(agg, dinv8, b8, w2)


def _head_body(agg_ref, dinv_ref, b_ref, wr_ref, br_ref, out_ref):
    di = dinv_ref[:, :1]
    full = jnp.concatenate([agg_ref[0], agg_ref[1]], axis=1)
    h = jnp.maximum(full * di + b_ref[0:1, :], 0.0)
    lg = jnp.dot(h, wr_ref[...], preferred_element_type=jnp.float32) + br_ref[0:1, :]
    m = jnp.max(lg, axis=1, keepdims=True)
    e = jnp.exp(lg - m)
    out_ref[...] = e / jnp.sum(e, axis=1, keepdims=True)


def _head(agg, dinv8, b8, wr, br8):
    return pl.pallas_call(
        _head_body,
        grid=(_NRB,),
        in_specs=[
            pl.BlockSpec((2, _RB, 128), lambda i: (0, i, 0)),
            pl.BlockSpec((_RB, 8), lambda i: (i, 0)),
            pl.BlockSpec((8, D_HID), lambda i: (0, 0)),
            pl.BlockSpec((D_HID, D_OUT), lambda i: (0, 0)),
            pl.BlockSpec((8, D_OUT), lambda i: (0, 0)),
        ],
        out_specs=pl.BlockSpec((_RB, D_OUT), lambda i: (i, 0)),
        out_shape=jax.ShapeDtypeStruct((N_PAD, D_OUT), jnp.float32),
    )(agg, dinv8, b8, wr, br8)


@jax.jit
def _run(x, edge_index, w1, b1, w2, b2, wr, br):
    src = edge_index[0]
    dst = edge_index[1]
    pad = E_PAD - E_EDGES
    srcp = jnp.concatenate([src, jnp.zeros((pad,), jnp.int32)])
    dstp = jnp.concatenate([dst, jnp.full((pad,), N_NODES, jnp.int32)])
    src1 = jnp.concatenate([srcp, srcp + N_PAD])
    dst1 = dstp

    dpad = DEG_E_PAD - E_EDGES
    dst2d = jnp.concatenate(
        [dst, jnp.full((dpad,), N_NODES, jnp.int32)]
    ).reshape(DEG_E_ROWS, 128)

    ones8 = jnp.ones((128, 8), jnp.float32)
    deg8 = _deg_kernel(ones8, dst2d)

    x_pad = jnp.concatenate([x, jnp.zeros((N_PAD - N_NODES, D_IN), jnp.float32)])
    tab1, dinv8 = _lin1(x_pad, w1, deg8)

    agg1 = _agg_kernel(tab1.reshape(2 * N_PAD, 128), src1, dst1).reshape(
        2, N_PAD, 128
    )

    b1_8 = jnp.broadcast_to(b1, (8, D_HID))
    tab2 = _mid(agg1, dinv8, b1_8, w2)

    agg2 = _agg_kernel(tab2.reshape(2 * N_PAD, 128), src1, dst1).reshape(
        2, N_PAD, 128
    )

    b2_8 = jnp.broadcast_to(b2, (8, D_HID))
    br_8 = jnp.broadcast_to(br, (8, D_OUT))
    out = _head(agg2, dinv8, b2_8, wr, br_8)
    return out[:N_NODES]


def kernel(x, edge_index, W1, b1, W2, b2, Wr, br):
    return _run(x, edge_index, W1, b1, W2, b2, Wr, br)


# speedup vs baseline: 21.1830x; 1.0123x over previous
import functools

import jax
import jax.numpy as jnp
from jax import lax
from jax.experimental import pallas as pl
from jax.experimental.pallas import tpu as pltpu
from jax.experimental.pallas import tpu_sc as plsc

N_NODES = 10000
N_PAD = 10240
E_EDGES = 160000
CHUNK = 72
CPT = 139
E_PAD = 16 * CPT * CHUNK
ACC_ROWS = 10112
ACC_RPT = ACC_ROWS // 16
DEG_CPT = 80
DEG_E_PAD = 16 * DEG_CPT * 128
DEG_E_ROWS = DEG_E_PAD // 128
D_IN = 256
D_HID = 256
D_OUT = 128

_MESH = plsc.VectorSubcoreMesh(
    core_axis_name="c", subcore_axis_name="s", num_cores=2, num_subcores=16
)


@functools.partial(
    pl.kernel,
    out_type=jax.ShapeDtypeStruct((N_PAD, 8), jnp.float32),
    mesh=_MESH,
    scratch_types=[
        pltpu.VMEM((128, 8), jnp.float32),
        pltpu.VMEM((DEG_CPT, 128), jnp.int32),
        pltpu.VMEM_SHARED((N_PAD, 8), jnp.float32),
        pltpu.SemaphoreType.DMA,
    ],
)
def _deg_kernel(ones_hbm, dst2d_hbm, out_hbm, ones_v, didx, acc_sh, sem):
    c = lax.axis_index("c")
    s = lax.axis_index("s")
    pltpu.sync_copy(ones_hbm, ones_v)
    pltpu.sync_copy(dst2d_hbm.at[pl.ds(s * DEG_CPT, DEG_CPT)], didx)
    rpt = N_PAD // 16
    for r in range(rpt // 128):
        pltpu.sync_copy(ones_v, acc_sh.at[pl.ds(s * rpt + r * 128, 128)])
    plsc.subcore_barrier()

    def body(k, carry):
        pltpu.async_copy(ones_v, acc_sh.at[didx.at[k]], sem, add=True)

        @pl.when(k >= 8)
        def _():
            pltpu.make_async_copy(ones_v, acc_sh.at[didx.at[0]], sem).wait()

        return carry

    lax.fori_loop(0, DEG_CPT, body, 0)
    for _ in range(8):
        pltpu.make_async_copy(ones_v, acc_sh.at[didx.at[0]], sem).wait()
    plsc.subcore_barrier()
    orow = N_PAD // 32
    ob = c * (N_PAD // 2) + s * orow
    pltpu.sync_copy(acc_sh.at[pl.ds(ob, orow)], out_hbm.at[pl.ds(ob, orow)])


@functools.partial(
    pl.kernel,
    out_type=jax.ShapeDtypeStruct((2 * N_PAD, 128), jnp.float32),
    mesh=_MESH,
    scratch_types=[
        pltpu.VMEM((6, CHUNK), jnp.int32),
        pltpu.VMEM((6, CHUNK), jnp.int32),
        pltpu.VMEM((5, CHUNK, 128), jnp.float32),
        pltpu.VMEM_SHARED((ACC_ROWS, 128), jnp.float32),
        pltpu.SemaphoreType.DMA,
        pltpu.SemaphoreType.DMA,
        pltpu.SemaphoreType.DMA,
        pltpu.SemaphoreType.DMA,
    ],
)
def _agg_kernel(
    table_hbm, src1_hbm, dst1_hbm, out_hbm, didxr, sidxr, rows, acc_sh,
    isem, gsem, ssem, nsem,
):
    c = lax.axis_index("c")
    s = lax.axis_index("s")
    b = s * ACC_RPT
    eb = s * (CPT * CHUNK)
    db = s * (CPT * CHUNK)
    tab_c = table_hbm.at[pl.ds(c * N_PAD, N_PAD)]
    ini = pltpu.async_copy(
        table_hbm.at[pl.ds(c * N_PAD + b, ACC_RPT)], acc_sh.at[pl.ds(b, ACC_RPT)],
        nsem,
    )
    for r in range(5):
        pltpu.async_copy(
            src1_hbm.at[pl.ds(eb + r * CHUNK, CHUNK)], sidxr.at[r], isem
        )
        pltpu.async_copy(
            dst1_hbm.at[pl.ds(db + r * CHUNK, CHUNK)], didxr.at[r], isem
        )
    for r in range(4):
        pltpu.make_async_copy(
            src1_hbm.at[pl.ds(eb, CHUNK)], sidxr.at[r], isem
        ).wait()
        pltpu.make_async_copy(
            dst1_hbm.at[pl.ds(db, CHUNK)], didxr.at[r], isem
        ).wait()
        pltpu.async_copy(tab_c.at[sidxr.at[r]], rows.at[r], gsem)
    ini.wait()
    plsc.subcore_barrier()

    def slot(k, carry):
        pk = lax.rem(k, 5)

        pltpu.make_async_copy(
            table_hbm.at[pl.ds(0, CHUNK)], rows.at[pk], gsem
        ).wait()
        pltpu.async_copy(rows.at[pk], acc_sh.at[didxr.at[lax.rem(k, 6)]], ssem,
                         add=True)

        @pl.when(k >= 1)
        def _():
            pltpu.make_async_copy(
                rows.at[pk], acc_sh.at[didxr.at[lax.rem(k, 6)]], ssem
            ).wait()

        @pl.when(k + 4 < CPT)
        def _():
            pltpu.make_async_copy(
                src1_hbm.at[pl.ds(eb, CHUNK)], sidxr.at[0], isem
            ).wait()
            pltpu.make_async_copy(
                dst1_hbm.at[pl.ds(db, CHUNK)], didxr.at[0], isem
            ).wait()
            pltpu.async_copy(
                tab_c.at[sidxr.at[lax.rem(k + 4, 6)]],
                rows.at[lax.rem(k + 4, 5)],
                gsem,
            )

        @pl.when(k + 5 < CPT)
        def _():
            pltpu.async_copy(
                src1_hbm.at[pl.ds(eb + (k + 5) * CHUNK, CHUNK)],
                sidxr.at[lax.rem(k + 5, 6)],
                isem,
            )
            pltpu.async_copy(
                dst1_hbm.at[pl.ds(db + (k + 5) * CHUNK, CHUNK)],
                didxr.at[lax.rem(k + 5, 6)],
                isem,
            )

        return carry

    lax.fori_loop(0, CPT, slot, 0)
    pltpu.make_async_copy(
        rows.at[lax.rem(CPT - 1, 5)], acc_sh.at[didxr.at[0]], ssem
    ).wait()
    plsc.subcore_barrier()
    pltpu.sync_copy(
        acc_sh.at[pl.ds(b, ACC_RPT)], out_hbm.at[pl.ds(c * N_PAD + b, ACC_RPT)]
    )


_RB = 1024
_NRB = N_PAD // _RB


def _lin1_body(x_ref, w_ref, deg_ref, tab_ref, dinv_ref):
    di8 = lax.rsqrt(deg_ref[...])
    dinv_ref[...] = di8
    h = jnp.dot(x_ref[...], w_ref[...], preferred_element_type=jnp.float32)
    h = h * di8[:, :1]
    tab_ref[0] = h[:, :128]
    tab_ref[1] = h[:, 128:]


def _lin1(x_pad, w1, deg8):
    return pl.pallas_call(
        _lin1_body,
        grid=(_NRB,),
        in_specs=[
            pl.BlockSpec((_RB, D_IN), lambda i: (i, 0)),
            pl.BlockSpec((D_IN, D_HID), lambda i: (0, 0)),
            pl.BlockSpec((_RB, 8), lambda i: (i, 0)),
        ],
        out_specs=[
            pl.BlockSpec((2, _RB, 128), lambda i: (0, i, 0)),
            pl.BlockSpec((_RB, 8), lambda i: (i, 0)),
        ],
        out_shape=[
            jax.ShapeDtypeStruct((2, N_PAD, 128), jnp.float32),
            jax.ShapeDtypeStruct((N_PAD, 8), jnp.float32),
        ],
    )(x_pad, w1, deg8)


def _mid_body(agg_ref, dinv_ref, b_ref, w_ref, out_ref):
    di = dinv_ref[:, :1]
    full = jnp.concatenate([agg_ref[0], agg_ref[1]], axis=1)
    h = jnp.maximum(full * di + b_ref[0:1, :], 0.0)
    o = jnp.dot(h, w_ref[...], preferred_element_type=jnp.float32) * di
    out_ref[0] = o[:, :128]
    out_ref[1] = o[:, 128:]


def _mid(agg, dinv8, b8, w2):
    return pl.pallas_call(
        _mid_body,
        grid=(_NRB,),
        in_specs=[
            pl.BlockSpec((2, _RB, 128), lambda i: (0, i, 0)),
            pl.BlockSpec((_RB, 8), lambda i: (i, 0)),
            pl.BlockSpec((8, D_HID), lambda i: (0, 0)),
            pl.BlockSpec((D_HID, D_HID), lambda i: (0, 0)),
        ],
        out_specs=pl.BlockSpec((2, _RB, 128), lambda i: (0, i, 0)),
        out_shape=jax.ShapeDtypeStruct((2, N_PAD, 128), jnp.float32),
    )(agg, dinv8, b8, w2)


def _head_body(agg_ref, dinv_ref, b_ref, wr_ref, br_ref, out_ref):
    di = dinv_ref[:, :1]
    full = jnp.concatenate([agg_ref[0], agg_ref[1]], axis=1)
    h = jnp.maximum(full * di + b_ref[0:1, :], 0.0)
    lg = jnp.dot(h, wr_ref[...], preferred_element_type=jnp.float32) + br_ref[0:1, :]
    m = jnp.max(lg, axis=1, keepdims=True)
    e = jnp.exp(lg - m)
    out_ref[...] = e / jnp.sum(e, axis=1, keepdims=True)


_HRB = 1000


def _head(agg, dinv8, b8, wr, br8):
    return pl.pallas_call(
        _head_body,
        grid=(N_NODES // _HRB,),
        in_specs=[
            pl.BlockSpec((2, _HRB, 128), lambda i: (0, i, 0)),
            pl.BlockSpec((_HRB, 8), lambda i: (i, 0)),
            pl.BlockSpec((8, D_HID), lambda i: (0, 0)),
            pl.BlockSpec((D_HID, D_OUT), lambda i: (0, 0)),
            pl.BlockSpec((8, D_OUT), lambda i: (0, 0)),
        ],
        out_specs=pl.BlockSpec((_HRB, D_OUT), lambda i: (i, 0)),
        out_shape=jax.ShapeDtypeStruct((N_NODES, D_OUT), jnp.float32),
    )(agg, dinv8, b8, wr, br8)


@jax.jit
def _run(x, edge_index, w1, b1, w2, b2, wr, br):
    src = edge_index[0]
    dst = edge_index[1]
    pad = E_PAD - E_EDGES
    srcp = jnp.concatenate([src, jnp.zeros((pad,), jnp.int32)])
    dstp = jnp.concatenate([dst, jnp.full((pad,), N_NODES, jnp.int32)])
    src1 = srcp
    dst1 = dstp

    dpad = DEG_E_PAD - E_EDGES
    dst2d = jnp.concatenate(
        [dst, jnp.full((dpad,), N_NODES, jnp.int32)]
    ).reshape(DEG_E_ROWS, 128)

    ones8 = jnp.ones((128, 8), jnp.float32)
    deg8 = _deg_kernel(ones8, dst2d)

    x_pad = jnp.concatenate([x, jnp.zeros((N_PAD - N_NODES, D_IN), jnp.float32)])
    tab1, dinv8 = _lin1(x_pad, w1, deg8)

    agg1 = _agg_kernel(tab1.reshape(2 * N_PAD, 128), src1, dst1).reshape(
        2, N_PAD, 128
    )

    b1_8 = jnp.broadcast_to(b1, (8, D_HID))
    tab2 = _mid(agg1, dinv8, b1_8, w2)

    agg2 = _agg_kernel(tab2.reshape(2 * N_PAD, 128), src1, dst1).reshape(
        2, N_PAD, 128
    )

    b2_8 = jnp.broadcast_to(b2, (8, D_HID))
    br_8 = jnp.broadcast_to(br, (8, D_OUT))
    return _head(agg2, dinv8, b2_8, wr, br_8)


def kernel(x, edge_index, W1, b1, W2, b2, Wr, br):
    return _run(x, edge_index, W1, b1, W2, b2, Wr, br)
